# baseline (speedup 1.0000x reference)
import jax
import jax.numpy as jnp
from jax.experimental import pallas as pl
from jax.experimental.pallas import tpu as pltpu

NCLS = 1000
BJ = 8


def _one_hot_body(xt_ref, o_ref):
    k = jax.lax.broadcasted_iota(jnp.int32, (BJ, NCLS, 1024), 1)
    o_ref[...] = (k == xt_ref[...][:, None, :]).astype(jnp.int32)


def kernel(x):
    n0, n1 = x.shape
    xt = x.T
    out = pl.pallas_call(
        _one_hot_body,
        grid=(pl.cdiv(n1, BJ),),
        in_specs=[pl.BlockSpec((BJ, n0), lambda j: (j, 0))],
        out_specs=pl.BlockSpec((BJ, NCLS, n0), lambda j: (j, 0, 0)),
        out_shape=jax.ShapeDtypeStruct((n1, NCLS, n0), jnp.int32),
        compiler_params=pltpu.CompilerParams(
            vmem_limit_bytes=120 * 1024 * 1024,
        ),
    )(xt)
    return out.transpose(2, 0, 1)


# speedup vs baseline: 1.0569x; 1.0569x over previous
import jax
import jax.numpy as jnp
from jax.experimental import pallas as pl

NCLS = 1000
BJ = 1


def _one_hot_body(xt_ref, o_ref):
    k = jax.lax.broadcasted_iota(jnp.int32, (BJ, NCLS, 1024), 1)
    o_ref[...] = (k == xt_ref[...]).astype(jnp.int32)


def kernel(x):
    n0, n1 = x.shape
    xt = x.T.reshape(n1, 1, n0)
    out = pl.pallas_call(
        _one_hot_body,
        grid=(n1 // BJ,),
        in_specs=[pl.BlockSpec((BJ, 1, n0), lambda j: (j, 0, 0))],
        out_specs=pl.BlockSpec((BJ, NCLS, n0), lambda j: (j, 0, 0)),
        out_shape=jax.ShapeDtypeStruct((n1, NCLS, n0), jnp.int32),
    )(xt)
    return out.transpose(2, 0, 1)


# speedup vs baseline: 1.1038x; 1.0445x over previous
import jax
import jax.numpy as jnp
from jax.experimental import pallas as pl

NCLS = 1000


def _one_hot_body(xt_ref, o_ref):
    j = pl.program_id(0)
    xall = xt_ref[...]
    m = jax.lax.broadcasted_iota(jnp.int32, xall.shape, 0) == j
    xr = jnp.sum(jnp.where(m, xall, 0), axis=0)[None, None, :]
    k = jax.lax.broadcasted_iota(jnp.int32, (1, NCLS, 1024), 1)
    o_ref[...] = (k == xr).astype(jnp.int32)


def kernel(x):
    n0, n1 = x.shape
    xt = x.T
    out = pl.pallas_call(
        _one_hot_body,
        grid=(n1,),
        in_specs=[pl.BlockSpec((n1, n0), lambda j: (0, 0))],
        out_specs=pl.BlockSpec((1, NCLS, n0), lambda j: (j, 0, 0)),
        out_shape=jax.ShapeDtypeStruct((n1, NCLS, n0), jnp.int32),
    )(xt)
    return out.transpose(2, 0, 1)
